# baseline (speedup 1.0000x reference)
import functools

import jax
import jax.numpy as jnp
from jax import lax
from jax.experimental import pallas as pl
from jax.experimental.pallas import tpu as pltpu
from jax.experimental.pallas import tpu_sc as plsc

_EMBED = 128
_BATCH = 16384
_MARGIN = 1.0
_LANES = 16
_NSEG = _EMBED // _LANES

_NC = 2
_NS = 16
_NW = _NC * _NS
_B_PER_W = _BATCH // _NW
_CHUNK = 64
_NCHUNK = _B_PER_W // _CHUNK


def _tec_kernel(pos_hbm, neg_hbm, ent_hbm, rel_hbm, out_hbm,
                idx_ph, idx_pr, idx_pt, idx_nh, idx_nr, idx_nt,
                ph0, pr0, pt0, nh0, nr0, nt0,
                ph1, pr1, pt1, nh1, nr1, nt1,
                out_v, sem):
    wid = lax.axis_index("s") * _NC + lax.axis_index("c")
    base0 = wid * _B_PER_W
    sl0 = pl.ds(base0, _B_PER_W)

    pltpu.sync_copy(pos_hbm.at[pl.ds(0, 1), sl0], idx_ph)
    pltpu.sync_copy(pos_hbm.at[pl.ds(1, 1), sl0], idx_pr)
    pltpu.sync_copy(pos_hbm.at[pl.ds(2, 1), sl0], idx_pt)
    pltpu.sync_copy(neg_hbm.at[pl.ds(0, 1), sl0], idx_nh)
    pltpu.sync_copy(neg_hbm.at[pl.ds(1, 1), sl0], idx_nr)
    pltpu.sync_copy(neg_hbm.at[pl.ds(2, 1), sl0], idx_nt)

    tables = (ent_hbm, rel_hbm, ent_hbm, ent_hbm, rel_hbm, ent_hbm)
    idxs = (idx_ph, idx_pr, idx_pt, idx_nh, idx_nr, idx_nt)
    bufsets = ((ph0, pr0, pt0, nh0, nr0, nt0),
               (ph1, pr1, pt1, nh1, nr1, nt1))

    def fire(ci):
        s = ci % 2
        return [pltpu.async_copy(
                    tab.at[idx.at[0, pl.ds(ci * _CHUNK, _CHUNK)]], buf, sem)
                for tab, idx, buf in zip(tables, idxs, bufsets[s])]

    lane = lax.broadcasted_iota(jnp.int32, (_LANES,), 0)

    def compute_chunk(s, acc0):
        ph, pr, pt, nh, nr, nt = bufsets[s]

        def row_diff(b):
            t = []
            for j in range(_NSEG):
                ds = pl.ds(j * _LANES, _LANES)
                pd = jnp.abs(ph[b, ds] + pr[b, ds] - pt[b, ds])
                nd = jnp.abs(nh[b, ds] + nr[b, ds] - nt[b, ds])
                t.append(nd - pd)
            while len(t) > 1:
                t = [x + y for x, y in zip(t[0::2], t[1::2])]
            return t[0]

        def pair_body(i, acc):
            b = 2 * i
            da = row_diff(b)
            db = row_diff(b + 1)
            m = jnp.where(
                lane < 8,
                da + da.at[lane ^ 8].get(mode="promise_in_bounds"),
                db + db.at[lane ^ 8].get(mode="promise_in_bounds"))
            for k in (1, 2, 4):
                m = m + m.at[lane ^ k].get(mode="promise_in_bounds")
            c = jnp.maximum(m + _MARGIN, 0.0)
            return acc + jnp.where((lane & 7) == 0, c, 0.0)

        return lax.fori_loop(0, _CHUNK // 2, pair_body, acc0)

    acc = jnp.zeros((_LANES,), jnp.float32)
    pending = fire(0)
    for ci in range(_NCHUNK):
        nxt = fire(ci + 1) if ci + 1 < _NCHUNK else None
        for cp in pending:
            cp.wait()
        acc = compute_chunk(ci % 2, acc)
        pending = nxt

    out_v[...] = acc
    pltpu.sync_copy(out_v, out_hbm.at[wid])


@jax.jit
def kernel(pos_exmpl, neg_exmpl, entity_emb, relation_emb):
    mesh = plsc.VectorSubcoreMesh(core_axis_name="c", subcore_axis_name="s")
    buf = pltpu.VMEM((_CHUNK, _EMBED), jnp.float32)
    run = functools.partial(
        pl.kernel,
        mesh=mesh,
        out_type=jax.ShapeDtypeStruct((_NW, _LANES), jnp.float32),
        scratch_types=(
            [pltpu.VMEM((1, _B_PER_W), jnp.int32)] * 6
            + [buf] * 12
            + [pltpu.VMEM((_LANES,), jnp.float32), pltpu.SemaphoreType.DMA]
        ),
    )(_tec_kernel)
    partials = run(pos_exmpl, neg_exmpl, entity_emb, relation_emb)
    return jnp.sum(partials)


# speedup vs baseline: 1.1333x; 1.1333x over previous
import functools

import jax
import jax.numpy as jnp
from jax import lax
from jax.experimental import pallas as pl
from jax.experimental.pallas import tpu as pltpu
from jax.experimental.pallas import tpu_sc as plsc

_EMBED = 128
_BATCH = 16384
_MARGIN = 1.0
_LANES = 16
_NSEG = _EMBED // _LANES

_NC = 2
_NS = 16
_NW = _NC * _NS
_B_PER_W = _BATCH // _NW
_CHUNK = 64
_NCHUNK = _B_PER_W // _CHUNK


def _tec_kernel(pos_hbm, neg_hbm, ent_hbm, rel_hbm, out_hbm,
                idx_ph, idx_pr, idx_pt, idx_nh, idx_nr, idx_nt,
                ph0, pr0, pt0, nh0, nr0, nt0,
                ph1, pr1, pt1, nh1, nr1, nt1,
                out_v, sem):
    wid = lax.axis_index("s") * _NC + lax.axis_index("c")
    base0 = wid * _B_PER_W
    sl0 = pl.ds(base0, _B_PER_W)

    idx_cps = [
        pltpu.async_copy(pos_hbm.at[pl.ds(0, 1), sl0], idx_ph, sem),
        pltpu.async_copy(pos_hbm.at[pl.ds(1, 1), sl0], idx_pr, sem),
        pltpu.async_copy(pos_hbm.at[pl.ds(2, 1), sl0], idx_pt, sem),
        pltpu.async_copy(neg_hbm.at[pl.ds(0, 1), sl0], idx_nh, sem),
        pltpu.async_copy(neg_hbm.at[pl.ds(1, 1), sl0], idx_nr, sem),
        pltpu.async_copy(neg_hbm.at[pl.ds(2, 1), sl0], idx_nt, sem),
    ]
    for cp in idx_cps:
        cp.wait()

    tables = (ent_hbm, rel_hbm, ent_hbm, ent_hbm, rel_hbm, ent_hbm)
    idxs = (idx_ph, idx_pr, idx_pt, idx_nh, idx_nr, idx_nt)
    bufsets = ((ph0, pr0, pt0, nh0, nr0, nt0),
               (ph1, pr1, pt1, nh1, nr1, nt1))

    def fire(ci):
        s = ci % 2
        return [pltpu.async_copy(
                    tab.at[idx.at[0, pl.ds(ci * _CHUNK, _CHUNK)]], buf, sem)
                for tab, idx, buf in zip(tables, idxs, bufsets[s])]

    lane = lax.broadcasted_iota(jnp.int32, (_LANES,), 0)

    def compute_chunk(s, acc0):
        ph, pr, pt, nh, nr, nt = bufsets[s]

        def row_body(b, acc):
            d = jnp.zeros((_LANES,), jnp.float32)
            for j in range(_NSEG):
                ds = pl.ds(j * _LANES, _LANES)
                pd = jnp.abs(ph[b, ds] + pr[b, ds] - pt[b, ds])
                nd = jnp.abs(nh[b, ds] + nr[b, ds] - nt[b, ds])
                d = d + (nd - pd)
            for k in (1, 2, 4, 8):
                d = d + d.at[lane ^ k].get(mode="promise_in_bounds")
            return acc + jnp.maximum(d + _MARGIN, 0.0)

        return lax.fori_loop(0, _CHUNK, row_body, acc0)

    acc = jnp.zeros((_LANES,), jnp.float32)
    pending = fire(0)
    for ci in range(_NCHUNK):
        nxt = fire(ci + 1) if ci + 1 < _NCHUNK else None
        for cp in pending:
            cp.wait()
        acc = compute_chunk(ci % 2, acc)
        pending = nxt

    out_v[...] = acc * (1.0 / _LANES)
    pltpu.sync_copy(out_v, out_hbm.at[wid])


@jax.jit
def kernel(pos_exmpl, neg_exmpl, entity_emb, relation_emb):
    mesh = plsc.VectorSubcoreMesh(core_axis_name="c", subcore_axis_name="s")
    buf = pltpu.VMEM((_CHUNK, _EMBED), jnp.float32)
    run = functools.partial(
        pl.kernel,
        mesh=mesh,
        out_type=jax.ShapeDtypeStruct((_NW, _LANES), jnp.float32),
        scratch_types=(
            [pltpu.VMEM((1, _B_PER_W), jnp.int32)] * 6
            + [buf] * 12
            + [pltpu.VMEM((_LANES,), jnp.float32), pltpu.SemaphoreType.DMA]
        ),
    )(_tec_kernel)
    partials = run(pos_exmpl, neg_exmpl, entity_emb, relation_emb)
    return jnp.sum(partials)


# speedup vs baseline: 1.1340x; 1.0006x over previous
import functools

import jax
import jax.numpy as jnp
from jax import lax
from jax.experimental import pallas as pl
from jax.experimental.pallas import tpu as pltpu
from jax.experimental.pallas import tpu_sc as plsc

_EMBED = 128
_BATCH = 16384
_MARGIN = 1.0
_LANES = 16
_NSEG = _EMBED // _LANES

_NC = 2
_NS = 16
_NW = _NC * _NS
_B_PER_W = _BATCH // _NW
_CHUNK = 64
_NCHUNK = _B_PER_W // _CHUNK


def _tec_kernel(pos_hbm, neg_hbm, ent_hbm, rel_hbm, out_hbm,
                idx_ph, idx_pr, idx_pt, idx_nh, idx_nr, idx_nt,
                ph0, pr0, pt0, nh0, nr0, nt0,
                ph1, pr1, pt1, nh1, nr1, nt1,
                out_v, sem):
    wid = lax.axis_index("s") * _NC + lax.axis_index("c")
    base0 = wid * _B_PER_W
    sl0 = pl.ds(base0, _B_PER_W)

    _H = 2 * _CHUNK
    srcs = ((pos_hbm, 0), (pos_hbm, 1), (pos_hbm, 2),
            (neg_hbm, 0), (neg_hbm, 1), (neg_hbm, 2))
    idxs6 = (idx_ph, idx_pr, idx_pt, idx_nh, idx_nr, idx_nt)
    head = [pltpu.async_copy(a.at[pl.ds(r, 1), pl.ds(base0, _H)],
                             idx.at[:, pl.ds(0, _H)], sem)
            for (a, r), idx in zip(srcs, idxs6)]
    tail = [pltpu.async_copy(a.at[pl.ds(r, 1), pl.ds(base0 + _H,
                                                     _B_PER_W - _H)],
                             idx.at[:, pl.ds(_H, _B_PER_W - _H)], sem)
            for (a, r), idx in zip(srcs, idxs6)]
    for cp in head:
        cp.wait()

    tables = (ent_hbm, rel_hbm, ent_hbm, ent_hbm, rel_hbm, ent_hbm)
    idxs = (idx_ph, idx_pr, idx_pt, idx_nh, idx_nr, idx_nt)
    bufsets = ((ph0, pr0, pt0, nh0, nr0, nt0),
               (ph1, pr1, pt1, nh1, nr1, nt1))

    def fire(ci):
        s = ci % 2
        return [pltpu.async_copy(
                    tab.at[idx.at[0, pl.ds(ci * _CHUNK, _CHUNK)]], buf, sem)
                for tab, idx, buf in zip(tables, idxs, bufsets[s])]

    lane = lax.broadcasted_iota(jnp.int32, (_LANES,), 0)

    def compute_chunk(s, acc0):
        ph, pr, pt, nh, nr, nt = bufsets[s]

        def row_body(b, acc):
            d = jnp.zeros((_LANES,), jnp.float32)
            for j in range(_NSEG):
                ds = pl.ds(j * _LANES, _LANES)
                pd = jnp.abs(ph[b, ds] + pr[b, ds] - pt[b, ds])
                nd = jnp.abs(nh[b, ds] + nr[b, ds] - nt[b, ds])
                d = d + (nd - pd)
            for k in (1, 2, 4, 8):
                d = d + d.at[lane ^ k].get(mode="promise_in_bounds")
            return acc + jnp.maximum(d + _MARGIN, 0.0)

        return lax.fori_loop(0, _CHUNK, row_body, acc0)

    acc = jnp.zeros((_LANES,), jnp.float32)
    pending = fire(0)
    for cp in tail:
        cp.wait()
    for ci in range(_NCHUNK):
        nxt = fire(ci + 1) if ci + 1 < _NCHUNK else None
        for cp in pending:
            cp.wait()
        acc = compute_chunk(ci % 2, acc)
        pending = nxt

    out_v[...] = acc * (1.0 / _LANES)
    pltpu.sync_copy(out_v, out_hbm.at[wid])


@jax.jit
def kernel(pos_exmpl, neg_exmpl, entity_emb, relation_emb):
    mesh = plsc.VectorSubcoreMesh(core_axis_name="c", subcore_axis_name="s")
    buf = pltpu.VMEM((_CHUNK, _EMBED), jnp.float32)
    run = functools.partial(
        pl.kernel,
        mesh=mesh,
        out_type=jax.ShapeDtypeStruct((_NW, _LANES), jnp.float32),
        scratch_types=(
            [pltpu.VMEM((1, _B_PER_W), jnp.int32)] * 6
            + [buf] * 12
            + [pltpu.VMEM((_LANES,), jnp.float32), pltpu.SemaphoreType.DMA]
        ),
    )(_tec_kernel)
    partials = run(pos_exmpl, neg_exmpl, entity_emb, relation_emb)
    return jnp.sum(partials)
